# Initial kernel scaffold; baseline (speedup 1.0000x reference)
import functools

import jax
import jax.numpy as jnp
from jax import lax
from jax.experimental import pallas as pl
from jax.experimental.pallas import tpu as pltpu
from jax.experimental.pallas import tpu_sc as plsc

_B = 2
_D = 64
_NSUB = 16
_K = 128
_EPAD = 2048

_mesh = plsc.VectorSubcoreMesh(core_axis_name="c", subcore_axis_name="s")


def _tc_mlp(xs, ws, b1, w2, b2, res=None, out2=(), block=1000):
    M = max(x.shape[0] for x in xs if x.ndim == 2)
    H = w2.shape[0]
    dout = w2.shape[1]
    nblk = M // block
    npb = (M // _B) // block

    def spec(a):
        if a.ndim == 3:
            return pl.BlockSpec((1, block, a.shape[2]),
                                lambda i: (i // npb, lax.rem(i, npb), 0))
        nb = a.shape[0] // block
        return pl.BlockSpec((block, a.shape[1]),
                            lambda i, nb=nb: (lax.rem(i, nb), 0))

    def wspec(w):
        return pl.BlockSpec(w.shape, lambda i: (0, 0))

    operands = []
    in_specs = []
    has_w = tuple(w is not None for w in ws)
    for x, w in zip(xs, ws):
        operands.append(x)
        in_specs.append(spec(x))
        if w is not None:
            operands.append(w)
            in_specs.append(wspec(w))
    b1r = b1.reshape(1, H)
    b2r = b2.reshape(1, dout)
    operands += [b1r, w2, b2r]
    in_specs += [wspec(b1r), wspec(w2), wspec(b2r)]
    if res is not None:
        operands.append(res)
        in_specs.append(spec(res))
    for w in out2:
        operands.append(w)
        in_specs.append(wspec(w))

    n2 = len(out2)

    def body(*refs):
        it = iter(refs)
        terms = []
        for hw in has_w:
            x = next(it)
            w = next(it) if hw else None
            terms.append((x, w))
        b1_, w2_, b2_ = next(it), next(it), next(it)
        res_ = next(it) if res is not None else None
        w2s = [next(it) for _ in range(n2)]
        outs = [next(it) for _ in range(1 + n2)]
        acc = jnp.zeros((block, H), jnp.float32) + b1_[...]
        for x, w in terms:
            v = x[...]
            if v.ndim == 3:
                v = v[0]
            if w is not None:
                acc += jnp.dot(v, w[...], preferred_element_type=jnp.float32)
            else:
                acc += v
        y = jnp.dot(jnp.maximum(acc, 0.0), w2_[...],
                    preferred_element_type=jnp.float32) + b2_[...]
        if res_ is not None:
            r = res_[...]
            y = y + (r[0] if r.ndim == 3 else r)
        outs[0][...] = y
        for o, w in zip(outs[1:], w2s):
            o[...] = jnp.dot(y, w[...], preferred_element_type=jnp.float32)

    out_shape = [jax.ShapeDtypeStruct((M, dout), jnp.float32)]
    out_specs = [pl.BlockSpec((block, dout), lambda i: (i, 0))]
    for w in out2:
        out_shape.append(jax.ShapeDtypeStruct((M, w.shape[1]), jnp.float32))
        out_specs.append(pl.BlockSpec((block, w.shape[1]), lambda i: (i, 0)))

    r = pl.pallas_call(
        body,
        grid=(nblk,),
        in_specs=in_specs,
        out_specs=out_specs,
        out_shape=out_shape,
    )(*operands)
    return r[0] if n2 == 0 else r


def _sc_gather_pair(tbl_a, tbl_b, idx_a, idx_b):
    _, _, d = tbl_a.shape
    ep = idx_a.shape[0]
    es = ep // _NSUB
    nfull = es // _K

    @functools.partial(
        pl.kernel,
        mesh=_mesh,
        out_type=[jax.ShapeDtypeStruct((_B, ep, d), jnp.float32),
                  jax.ShapeDtypeStruct((_B, ep, d), jnp.float32)],
        scratch_types=[pltpu.VMEM((_K,), jnp.int32),
                       pltpu.VMEM((_K,), jnp.int32),
                       pltpu.VMEM((_K, d), jnp.float32),
                       pltpu.VMEM((_K, d), jnp.float32),
                       pltpu.SemaphoreType.DMA,
                       pltpu.SemaphoreType.DMA],
    )
    def k(ta, tb, ia, ib, oa, ob, va, vb, ra, rb, sa, sb):
        c = lax.axis_index("c")
        s = lax.axis_index("s")
        base = s * es

        @pl.loop(0, nfull)
        def _(i):
            off = base + i * _K
            pltpu.sync_copy(ia.at[pl.ds(off, _K)], va)
            pltpu.sync_copy(ib.at[pl.ds(off, _K)], vb)
            ca = pltpu.async_copy(ta.at[c].at[va], ra, sa)
            cb = pltpu.async_copy(tb.at[c].at[vb], rb, sb)
            ca.wait()
            cb.wait()
            pltpu.sync_copy(ra, oa.at[c].at[pl.ds(off, _K)])
            pltpu.sync_copy(rb, ob.at[c].at[pl.ds(off, _K)])

    return k(tbl_a, tbl_b, idx_a, idx_b)


def _sc_scatter_add(vals, idx, chunk, passes):
    _, ep, d = vals.shape
    es = ep // _NSUB
    nfull = es // _K
    sz = (chunk + 16) // _NSUB
    share = chunk // _NSUB
    zeros = jnp.zeros((sz, d), jnp.float32)

    @functools.partial(
        pl.kernel,
        mesh=_mesh,
        out_type=jax.ShapeDtypeStruct((_B, passes * chunk, d), jnp.float32),
        scratch_types=[pltpu.VMEM((_K,), jnp.int32),
                       pltpu.VMEM((_K,), jnp.int32),
                       pltpu.VMEM((_K, d), jnp.float32),
                       pltpu.VMEM_SHARED((chunk + 16, d), jnp.float32)],
    )
    def k(v_hbm, i_hbm, z_hbm, o_hbm, iv, lv, vv, acc):
        c = lax.axis_index("c")
        s = lax.axis_index("s")
        base = s * es
        for p in range(passes):
            off0 = p * chunk
            pltpu.sync_copy(z_hbm, acc.at[pl.ds(s * sz, sz)])
            plsc.subcore_barrier()

            @pl.loop(0, nfull)
            def _(i):
                off = base + i * _K
                pltpu.sync_copy(i_hbm.at[pl.ds(off, _K)], iv)
                pltpu.sync_copy(v_hbm.at[c].at[pl.ds(off, _K)], vv)
                for j in range(_K // 16):
                    w = iv[pl.ds(j * 16, 16)]
                    loc = w - off0
                    ok = (loc >= 0) & (loc < chunk)
                    lv[pl.ds(j * 16, 16)] = jnp.where(ok, loc, chunk)
                pltpu.sync_copy(vv, acc.at[lv], add=True)

            plsc.subcore_barrier()
            pltpu.sync_copy(acc.at[pl.ds(s * share, share)],
                            o_hbm.at[c].at[pl.ds(off0 + s * share, share)])
            if p + 1 < passes:
                plsc.subcore_barrier()

    return k(vals, idx, zeros)


def _pad_rows(a, n):
    return jnp.concatenate(
        [a, jnp.zeros((n - a.shape[0],) + a.shape[1:], a.dtype)])


def _pad_idx(a, n, fill):
    return jnp.concatenate(
        [a.astype(jnp.int32), jnp.full((n - a.shape[0],), fill, jnp.int32)])


def _ceil_to(n, g):
    return -(-n // g) * g


def kernel(pndata, pn_coords, rn_coords, p2r_src, p2r_dst, p2r_edata,
           r2r_src, r2r_dst, r2r_edata, r2p_src, r2p_dst, r2p_edata, params):
    B, NP, DIN = pndata.shape
    NR = rn_coords.shape[0]
    E1, E2, E3 = p2r_src.shape[0], r2r_src.shape[0], r2p_src.shape[0]
    C = pn_coords.shape[1]
    E1p, E2p, E3p = (_ceil_to(e, _EPAD) for e in (E1, E2, E3))
    STEPS = 3

    p = params
    W1pn, b1pn, W2pn, b2pn = p['enc_pn']
    W1rn, b1rn, W2rn, b2rn = p['enc_rn']
    W1e1, b1e1, W2e1, b2e1 = p['enc_e']
    W1m1, b1m1, W2m1, b2m1 = p['enc_msg']
    W1n1, b1n1, W2n1, b2n1 = p['enc_node']
    W1e2, b1e2, W2e2, b2e2 = p['proc_e']
    W1m2, b1m2, W2m2, b2m2 = p['proc_msg']
    W1n2, b1n2, W2n2, b2n2 = p['proc_node']
    W1rd, b1rd, W2rd, b2rd = p['dec_rn']
    W1e3, b1e3, W2e3, b2e3 = p['dec_e']
    W1m3, b1m3, W2m3, b2m3 = p['dec_msg']
    W1n3, b1n3, W2n3, b2n3 = p['dec_node']
    W1do, b1do, W2do, b2do = p['dec_out']

    We1, Ws1, Wd1 = W1m1[:_D], W1m1[_D:2 * _D], W1m1[2 * _D:]
    We2, Ws2, Wd2 = W1m2[:_D], W1m2[_D:2 * _D], W1m2[2 * _D:]
    We3, Ws3, Wd3 = W1m3[:_D], W1m3[_D:2 * _D], W1m3[2 * _D:]
    Wa1, Wb1 = W1n1[:_D], W1n1[_D:]
    Wa2, Wb2 = W1n2[:_D], W1n2[_D:]
    Wa3, Wb3 = W1n3[:_D], W1n3[_D:]

    ed1 = _pad_rows(p2r_edata, E1p)
    ed2 = _pad_rows(r2r_edata, E2p)
    ed3 = _pad_rows(r2p_edata, E3p)
    src1 = _pad_idx(p2r_src, E1p, 0)
    dst1g = _pad_idx(p2r_dst, E1p, 0)
    dst1s = _pad_idx(p2r_dst, E1p, NR)
    src2 = _pad_idx(r2r_src, E2p, 0)
    dst2g = _pad_idx(r2r_dst, E2p, 0)
    dst2s = _pad_idx(r2r_dst, E2p, NR)
    src3 = _pad_idx(r2p_src, E3p, 0)
    dst3g = _pad_idx(r2p_dst, E3p, 0)
    dst3s = _pad_idx(r2p_dst, E3p, NP)

    CH_R = 5008
    CH_P = 25600

    pn, srcT1, dstT3 = _tc_mlp(
        [pn_coords, pndata.reshape(B * NP, DIN)], [W1pn[:C], W1pn[C:]],
        b1pn, W2pn, b2pn, out2=(Ws1, Wd3))
    rn0, dstT1 = _tc_mlp([rn_coords], [W1rn], b1rn, W2rn, b2rn, out2=(Wd1,))
    e1, eT1 = _tc_mlp([ed1], [W1e1], b1e1, W2e1, b2e1, out2=(We1,), block=2048)

    g_a, g_b = _sc_gather_pair(
        srcT1.reshape(B, NP, _D),
        jnp.broadcast_to(dstT1[None], (B, NR, _D)) + 0.0,
        src1, dst1g)
    e1n = _tc_mlp([eT1, g_a.reshape(B * E1p, _D), g_b.reshape(B * E1p, _D)],
                  [None, None, None], b1m1, W2m1, b2m1, res=e1, block=2048)
    agg = _sc_scatter_add(e1n.reshape(B, E1p, _D), dst1s, CH_R, 1)
    rn, srcTp, dstTp = _tc_mlp(
        [rn0, agg], [Wa1, Wb1], b1n1, W2n1, b2n1, res=rn0, out2=(Ws2, Wd2))

    e2, eT2 = _tc_mlp([ed2], [W1e2], b1e2, W2e2, b2e2, out2=(We2,), block=2048)
    for t in range(STEPS):
        g_a, g_b = _sc_gather_pair(srcTp.reshape(B, NR, _D),
                                   dstTp.reshape(B, NR, _D), src2, dst2g)
        ga2 = g_a.reshape(B * E2p, _D)
        gb2 = g_b.reshape(B * E2p, _D)
        if t == 0:
            e2 = _tc_mlp([eT2, ga2, gb2], [None, None, None],
                         b1m2, W2m2, b2m2, res=e2, block=2048)
        else:
            e2 = _tc_mlp([e2, ga2, gb2], [We2, None, None],
                         b1m2, W2m2, b2m2, res=e2, block=2048)
        agg = _sc_scatter_add(e2.reshape(B, E2p, _D), dst2s, CH_R, 1)
        out2 = (Ws2, Wd2) if t + 1 < STEPS else ()
        r = _tc_mlp([rn, agg], [Wa2, Wb2], b1n2, W2n2, b2n2, res=rn, out2=out2)
        if out2:
            rn, srcTp, dstTp = r
        else:
            rn = r

    rnd, srcT3 = _tc_mlp([rn], [W1rd], b1rd, W2rd, b2rd, out2=(Ws3,))
    e3, eT3 = _tc_mlp([ed3], [W1e3], b1e3, W2e3, b2e3, out2=(We3,), block=2048)
    g_a, g_b = _sc_gather_pair(srcT3.reshape(B, NR, _D),
                               dstT3.reshape(B, NP, _D), src3, dst3g)
    e3n = _tc_mlp([eT3, g_a.reshape(B * E3p, _D), g_b.reshape(B * E3p, _D)],
                  [None, None, None], b1m3, W2m3, b2m3, res=e3, block=2048)
    aggp = _sc_scatter_add(e3n.reshape(B, E3p, _D), dst3s, CH_P, 2)
    pn1 = _tc_mlp([pn, aggp], [Wa3, Wb3], b1n3, W2n3, b2n3, res=pn)
    out = _tc_mlp([pn1], [W1do], b1do, W2do, b2do)
    return out.reshape(B, NP, -1)


# speedup vs baseline: 8.9399x; 8.9399x over previous
import functools

import jax
import jax.numpy as jnp
from jax import lax
from jax.experimental import pallas as pl
from jax.experimental.pallas import tpu as pltpu
from jax.experimental.pallas import tpu_sc as plsc

_B = 2
_D = 64
_NSUB = 16
_K = 128
_EPAD = 2048

_mesh = plsc.VectorSubcoreMesh(core_axis_name="c", subcore_axis_name="s")
_sc_params = pltpu.CompilerParams(use_tc_tiling_on_sc=False)


def _tc_mlp(xs, ws, b1, w2, b2, res=None, out2=(), block=1000, nrows=None):
    M = nrows or max(x.shape[0] for x in xs if x.ndim == 2)
    H = w2.shape[0]
    dout = w2.shape[1]
    nblk = M // block
    npb = (M // _B) // block

    def spec(a):
        if a.ndim == 3:
            return pl.BlockSpec((1, block, a.shape[2]),
                                lambda i: (i // npb, lax.rem(i, npb), 0))
        nb = a.shape[0] // block
        return pl.BlockSpec((block, a.shape[1]),
                            lambda i, nb=nb: (lax.rem(i, nb), 0))

    def wspec(w):
        return pl.BlockSpec(w.shape, lambda i: (0, 0))

    operands = []
    in_specs = []
    has_w = tuple(w is not None for w in ws)
    for x, w in zip(xs, ws):
        operands.append(x)
        in_specs.append(spec(x))
        if w is not None:
            operands.append(w)
            in_specs.append(wspec(w))
    b1r = b1.reshape(1, H)
    b2r = b2.reshape(1, dout)
    operands += [b1r, w2, b2r]
    in_specs += [wspec(b1r), wspec(w2), wspec(b2r)]
    if res is not None:
        operands.append(res)
        in_specs.append(spec(res))
    for w in out2:
        operands.append(w)
        in_specs.append(wspec(w))

    n2 = len(out2)

    def body(*refs):
        it = iter(refs)
        terms = []
        for hw in has_w:
            x = next(it)
            w = next(it) if hw else None
            terms.append((x, w))
        b1_, w2_, b2_ = next(it), next(it), next(it)
        res_ = next(it) if res is not None else None
        w2s = [next(it) for _ in range(n2)]
        outs = [next(it) for _ in range(1 + n2)]
        acc = jnp.zeros((block, H), jnp.float32) + b1_[...]
        for x, w in terms:
            v = x[...]
            if v.ndim == 3:
                v = v[0]
            if w is not None:
                acc += jnp.dot(v, w[...], preferred_element_type=jnp.float32)
            else:
                acc += v
        y = jnp.dot(jnp.maximum(acc, 0.0), w2_[...],
                    preferred_element_type=jnp.float32) + b2_[...]
        if res_ is not None:
            r = res_[...]
            y = y + (r[0] if r.ndim == 3 else r)
        outs[0][...] = y
        for o, w in zip(outs[1:], w2s):
            o[...] = jnp.dot(y, w[...], preferred_element_type=jnp.float32)

    out_shape = [jax.ShapeDtypeStruct((M, dout), jnp.float32)]
    out_specs = [pl.BlockSpec((block, dout), lambda i: (i, 0))]
    for w in out2:
        out_shape.append(jax.ShapeDtypeStruct((M, w.shape[1]), jnp.float32))
        out_specs.append(pl.BlockSpec((block, w.shape[1]), lambda i: (i, 0)))

    r = pl.pallas_call(
        body,
        grid=(nblk,),
        in_specs=in_specs,
        out_specs=out_specs,
        out_shape=out_shape,
    )(*operands)
    return r[0] if n2 == 0 else r


def _sc_gather_pair(tbl_a, tbl_b, idx_a, idx_b):
    _, _, d = tbl_a.shape
    ep = idx_a.shape[0]
    es = ep // _NSUB
    nfull = es // _K

    @functools.partial(
        pl.kernel,
        mesh=_mesh,
        out_type=[jax.ShapeDtypeStruct((_B, ep, d), jnp.float32),
                  jax.ShapeDtypeStruct((_B, ep, d), jnp.float32)],
        scratch_types=[pltpu.VMEM((_K,), jnp.int32),
                       pltpu.VMEM((_K,), jnp.int32),
                       pltpu.VMEM((_K, d), jnp.float32),
                       pltpu.VMEM((_K, d), jnp.float32),
                       pltpu.SemaphoreType.DMA,
                       pltpu.SemaphoreType.DMA],
        compiler_params=_sc_params,
    )
    def k(ta, tb, ia, ib, oa, ob, va, vb, ra, rb, sa, sb):
        c = lax.axis_index("c")
        s = lax.axis_index("s")
        base = s * es

        @pl.loop(0, nfull)
        def _(i):
            off = base + i * _K
            pltpu.sync_copy(ia.at[pl.ds(off, _K)], va)
            pltpu.sync_copy(ib.at[pl.ds(off, _K)], vb)
            ca = pltpu.async_copy(ta.at[c].at[va], ra, sa)
            cb = pltpu.async_copy(tb.at[c].at[vb], rb, sb)
            ca.wait()
            cb.wait()
            pltpu.sync_copy(ra, oa.at[c].at[pl.ds(off, _K)])
            pltpu.sync_copy(rb, ob.at[c].at[pl.ds(off, _K)])

    return k(tbl_a, tbl_b, idx_a, idx_b)


def _sc_scatter_add(vals, idx, chunk, passes):
    _, ep, d = vals.shape
    es = ep // _NSUB
    nfull = es // _K
    sz = (chunk + 16) // _NSUB
    share = chunk // _NSUB
    zeros = jnp.zeros((sz, d), jnp.float32)

    @functools.partial(
        pl.kernel,
        mesh=_mesh,
        out_type=jax.ShapeDtypeStruct((_B, passes * chunk, d), jnp.float32),
        scratch_types=[pltpu.VMEM((_K,), jnp.int32),
                       pltpu.VMEM((_K,), jnp.int32),
                       pltpu.VMEM((_K, d), jnp.float32),
                       pltpu.VMEM_SHARED((chunk + 16, d), jnp.float32)],
        compiler_params=_sc_params,
    )
    def k(v_hbm, i_hbm, z_hbm, o_hbm, iv, lv, vv, acc):
        c = lax.axis_index("c")
        s = lax.axis_index("s")
        base = s * es
        for p in range(passes):
            off0 = p * chunk
            pltpu.sync_copy(z_hbm, acc.at[pl.ds(s * sz, sz)])
            plsc.subcore_barrier()

            @pl.loop(0, nfull)
            def _(i):
                off = base + i * _K
                pltpu.sync_copy(i_hbm.at[pl.ds(off, _K)], iv)
                pltpu.sync_copy(v_hbm.at[c].at[pl.ds(off, _K)], vv)
                for j in range(_K // 16):
                    w = iv[pl.ds(j * 16, 16)]
                    loc = w - off0
                    ok = (loc >= 0) & (loc < chunk)
                    lv[pl.ds(j * 16, 16)] = jnp.where(ok, loc, chunk)
                pltpu.sync_copy(vv, acc.at[lv], add=True)

            plsc.subcore_barrier()
            pltpu.sync_copy(acc.at[pl.ds(s * share, share)],
                            o_hbm.at[c].at[pl.ds(off0 + s * share, share)])
            if p + 1 < passes:
                plsc.subcore_barrier()

    return k(vals, idx, zeros)


def _pad_rows(a, n):
    return jnp.concatenate(
        [a, jnp.zeros((n - a.shape[0],) + a.shape[1:], a.dtype)])


def _pad_idx(a, n, fill):
    return jnp.concatenate(
        [a.astype(jnp.int32), jnp.full((n - a.shape[0],), fill, jnp.int32)])


def _ceil_to(n, g):
    return -(-n // g) * g


def kernel(pndata, pn_coords, rn_coords, p2r_src, p2r_dst, p2r_edata,
           r2r_src, r2r_dst, r2r_edata, r2p_src, r2p_dst, r2p_edata, params):
    B, NP, DIN = pndata.shape
    NR = rn_coords.shape[0]
    E1, E2, E3 = p2r_src.shape[0], r2r_src.shape[0], r2p_src.shape[0]
    C = pn_coords.shape[1]
    E1p, E2p, E3p = (_ceil_to(e, _EPAD) for e in (E1, E2, E3))
    STEPS = 3

    p = params
    W1pn, b1pn, W2pn, b2pn = p['enc_pn']
    W1rn, b1rn, W2rn, b2rn = p['enc_rn']
    W1e1, b1e1, W2e1, b2e1 = p['enc_e']
    W1m1, b1m1, W2m1, b2m1 = p['enc_msg']
    W1n1, b1n1, W2n1, b2n1 = p['enc_node']
    W1e2, b1e2, W2e2, b2e2 = p['proc_e']
    W1m2, b1m2, W2m2, b2m2 = p['proc_msg']
    W1n2, b1n2, W2n2, b2n2 = p['proc_node']
    W1rd, b1rd, W2rd, b2rd = p['dec_rn']
    W1e3, b1e3, W2e3, b2e3 = p['dec_e']
    W1m3, b1m3, W2m3, b2m3 = p['dec_msg']
    W1n3, b1n3, W2n3, b2n3 = p['dec_node']
    W1do, b1do, W2do, b2do = p['dec_out']

    We1, Ws1, Wd1 = W1m1[:_D], W1m1[_D:2 * _D], W1m1[2 * _D:]
    We2, Ws2, Wd2 = W1m2[:_D], W1m2[_D:2 * _D], W1m2[2 * _D:]
    We3, Ws3, Wd3 = W1m3[:_D], W1m3[_D:2 * _D], W1m3[2 * _D:]
    Wa1, Wb1 = W1n1[:_D], W1n1[_D:]
    Wa2, Wb2 = W1n2[:_D], W1n2[_D:]
    Wa3, Wb3 = W1n3[:_D], W1n3[_D:]

    ed1 = _pad_rows(p2r_edata, E1p)
    ed2 = _pad_rows(r2r_edata, E2p)
    ed3 = _pad_rows(r2p_edata, E3p)
    src1 = _pad_idx(p2r_src, E1p, 0)
    dst1g = _pad_idx(p2r_dst, E1p, 0)
    dst1s = _pad_idx(p2r_dst, E1p, NR)
    src2 = _pad_idx(r2r_src, E2p, 0)
    dst2g = _pad_idx(r2r_dst, E2p, 0)
    dst2s = _pad_idx(r2r_dst, E2p, NR)
    src3 = _pad_idx(r2p_src, E3p, 0)
    dst3g = _pad_idx(r2p_dst, E3p, 0)
    dst3s = _pad_idx(r2p_dst, E3p, NP)

    CH_R = 5008
    CH_P = 25600

    pn, srcT1, dstT3 = _tc_mlp(
        [pn_coords, pndata.reshape(B * NP, DIN)], [W1pn[:C], W1pn[C:]],
        b1pn, W2pn, b2pn, out2=(Ws1, Wd3))
    rn0, dstT1 = _tc_mlp([rn_coords], [W1rn], b1rn, W2rn, b2rn, out2=(Wd1,))
    e1, eT1 = _tc_mlp([ed1], [W1e1], b1e1, W2e1, b2e1, out2=(We1,), block=2048)

    g_a, g_b = _sc_gather_pair(
        srcT1.reshape(B, NP, _D),
        jnp.broadcast_to(dstT1[None], (B, NR, _D)) + 0.0,
        src1, dst1g)
    e1n = _tc_mlp([eT1, g_a.reshape(B * E1p, _D), g_b.reshape(B * E1p, _D)],
                  [None, None, None], b1m1, W2m1, b2m1, res=e1, block=2048)
    agg = _sc_scatter_add(e1n.reshape(B, E1p, _D), dst1s, CH_R, 1)
    rn, srcTp, dstTp = _tc_mlp(
        [rn0, agg], [Wa1, Wb1], b1n1, W2n1, b2n1, res=rn0, out2=(Ws2, Wd2),
        nrows=B * NR)

    e2, eT2 = _tc_mlp([ed2], [W1e2], b1e2, W2e2, b2e2, out2=(We2,), block=2048)
    for t in range(STEPS):
        g_a, g_b = _sc_gather_pair(srcTp.reshape(B, NR, _D),
                                   dstTp.reshape(B, NR, _D), src2, dst2g)
        ga2 = g_a.reshape(B * E2p, _D)
        gb2 = g_b.reshape(B * E2p, _D)
        if t == 0:
            e2 = _tc_mlp([eT2, ga2, gb2], [None, None, None],
                         b1m2, W2m2, b2m2, res=e2, block=2048)
        else:
            e2 = _tc_mlp([e2, ga2, gb2], [We2, None, None],
                         b1m2, W2m2, b2m2, res=e2, block=2048)
        agg = _sc_scatter_add(e2.reshape(B, E2p, _D), dst2s, CH_R, 1)
        out2 = (Ws2, Wd2) if t + 1 < STEPS else ()
        r = _tc_mlp([rn, agg], [Wa2, Wb2], b1n2, W2n2, b2n2, res=rn,
                    out2=out2, nrows=B * NR)
        if out2:
            rn, srcTp, dstTp = r
        else:
            rn = r

    rnd, srcT3 = _tc_mlp([rn], [W1rd], b1rd, W2rd, b2rd, out2=(Ws3,))
    e3, eT3 = _tc_mlp([ed3], [W1e3], b1e3, W2e3, b2e3, out2=(We3,), block=2048)
    g_a, g_b = _sc_gather_pair(srcT3.reshape(B, NR, _D),
                               dstT3.reshape(B, NP, _D), src3, dst3g)
    e3n = _tc_mlp([eT3, g_a.reshape(B * E3p, _D), g_b.reshape(B * E3p, _D)],
                  [None, None, None], b1m3, W2m3, b2m3, res=e3, block=2048)
    aggp = _sc_scatter_add(e3n.reshape(B, E3p, _D), dst3s, CH_P, 2)
    pn1 = _tc_mlp([pn, aggp], [Wa3, Wb3], b1n3, W2n3, b2n3, res=pn,
                  nrows=B * NP)
    out = _tc_mlp([pn1], [W1do], b1do, W2do, b2do)
    return out.reshape(B, NP, -1)


# speedup vs baseline: 9.5890x; 1.0726x over previous
import functools

import jax
import jax.numpy as jnp
from jax import lax
from jax.experimental import pallas as pl
from jax.experimental.pallas import tpu as pltpu
from jax.experimental.pallas import tpu_sc as plsc

_B = 2
_D = 64
_NSUB = 16
_K = 128
_EPAD = 2048

_mesh = plsc.VectorSubcoreMesh(core_axis_name="c", subcore_axis_name="s")
_sc_params = pltpu.CompilerParams(use_tc_tiling_on_sc=False)


def _tc_mlp(xs, ws, b1, w2, b2, res=None, out2=(), block=1000, nrows=None):
    M = nrows or max(x.shape[0] for x in xs if x.ndim == 2)
    H = w2.shape[0]
    dout = w2.shape[1]
    nblk = M // block
    npb = (M // _B) // block

    def spec(a):
        if a.ndim == 3:
            return pl.BlockSpec((1, block, a.shape[2]),
                                lambda i: (i // npb, lax.rem(i, npb), 0))
        nb = a.shape[0] // block
        return pl.BlockSpec((block, a.shape[1]),
                            lambda i, nb=nb: (lax.rem(i, nb), 0))

    def wspec(w):
        return pl.BlockSpec(w.shape, lambda i: (0, 0))

    operands = []
    in_specs = []
    has_w = tuple(w is not None for w in ws)
    for x, w in zip(xs, ws):
        operands.append(x)
        in_specs.append(spec(x))
        if w is not None:
            operands.append(w)
            in_specs.append(wspec(w))
    b1r = b1.reshape(1, H)
    b2r = b2.reshape(1, dout)
    operands += [b1r, w2, b2r]
    in_specs += [wspec(b1r), wspec(w2), wspec(b2r)]
    if res is not None:
        operands.append(res)
        in_specs.append(spec(res))
    for w in out2:
        operands.append(w)
        in_specs.append(wspec(w))

    n2 = len(out2)

    def body(*refs):
        it = iter(refs)
        terms = []
        for hw in has_w:
            x = next(it)
            w = next(it) if hw else None
            terms.append((x, w))
        b1_, w2_, b2_ = next(it), next(it), next(it)
        res_ = next(it) if res is not None else None
        w2s = [next(it) for _ in range(n2)]
        outs = [next(it) for _ in range(1 + n2)]
        acc = jnp.zeros((block, H), jnp.float32) + b1_[...]
        for x, w in terms:
            v = x[...]
            if v.ndim == 3:
                v = v[0]
            if w is not None:
                acc += jnp.dot(v, w[...], preferred_element_type=jnp.float32)
            else:
                acc += v
        y = jnp.dot(jnp.maximum(acc, 0.0), w2_[...],
                    preferred_element_type=jnp.float32) + b2_[...]
        if res_ is not None:
            r = res_[...]
            y = y + (r[0] if r.ndim == 3 else r)
        outs[0][...] = y
        for o, w in zip(outs[1:], w2s):
            o[...] = jnp.dot(y, w[...], preferred_element_type=jnp.float32)

    out_shape = [jax.ShapeDtypeStruct((M, dout), jnp.float32)]
    out_specs = [pl.BlockSpec((block, dout), lambda i: (i, 0))]
    for w in out2:
        out_shape.append(jax.ShapeDtypeStruct((M, w.shape[1]), jnp.float32))
        out_specs.append(pl.BlockSpec((block, w.shape[1]), lambda i: (i, 0)))

    r = pl.pallas_call(
        body,
        grid=(nblk,),
        in_specs=in_specs,
        out_specs=out_specs,
        out_shape=out_shape,
    )(*operands)
    return r[0] if n2 == 0 else r


_NSLOT = 4


def _sc_gather_pair(tbl_a, tbl_b, idx_a, idx_b):
    _, _, d = tbl_a.shape
    ep = idx_a.shape[0]
    es = ep // _NSUB
    nfull = es // _K

    @functools.partial(
        pl.kernel,
        mesh=_mesh,
        out_type=[jax.ShapeDtypeStruct((_B, ep, d), jnp.float32),
                  jax.ShapeDtypeStruct((_B, ep, d), jnp.float32)],
        scratch_types=[pltpu.VMEM((es,), jnp.int32),
                       pltpu.VMEM((es,), jnp.int32),
                       pltpu.VMEM((_NSLOT, _K, d), jnp.float32),
                       pltpu.VMEM((_NSLOT, _K, d), jnp.float32),
                       pltpu.SemaphoreType.DMA((_NSLOT,)),
                       pltpu.SemaphoreType.DMA((_NSLOT,)),
                       pltpu.SemaphoreType.DMA((_NSLOT,)),
                       pltpu.SemaphoreType.DMA((_NSLOT,))],
        compiler_params=_sc_params,
    )
    def k(ta, tb, ia, ib, oa, ob, iva, ivb, ra, rb, sga, sgb, soa, sob):
        c = lax.axis_index("c")
        s = lax.axis_index("s")
        base = s * es
        pltpu.sync_copy(ia.at[pl.ds(base, es)], iva)
        pltpu.sync_copy(ib.at[pl.ds(base, es)], ivb)

        def issue_gather(i, r):
            pltpu.async_copy(ta.at[c].at[iva.at[pl.ds(i * _K, _K)]],
                             ra.at[r], sga.at[r])
            pltpu.async_copy(tb.at[c].at[ivb.at[pl.ds(i * _K, _K)]],
                             rb.at[r], sgb.at[r])

        def wait_gather(r):
            pltpu.make_async_copy(ta.at[c].at[pl.ds(0, _K)], ra.at[r],
                                  sga.at[r]).wait()
            pltpu.make_async_copy(tb.at[c].at[pl.ds(0, _K)], rb.at[r],
                                  sgb.at[r]).wait()

        def issue_out(i, r):
            off = base + i * _K
            pltpu.async_copy(ra.at[r], oa.at[c].at[pl.ds(off, _K)], soa.at[r])
            pltpu.async_copy(rb.at[r], ob.at[c].at[pl.ds(off, _K)], sob.at[r])

        def wait_out(r):
            pltpu.make_async_copy(ta.at[c].at[pl.ds(0, _K)], ra.at[r],
                                  soa.at[r]).wait()
            pltpu.make_async_copy(tb.at[c].at[pl.ds(0, _K)], rb.at[r],
                                  sob.at[r]).wait()

        for i in range(min(_NSLOT - 1, nfull)):
            issue_gather(i, i % _NSLOT)
        for i in range(nfull):
            r = i % _NSLOT
            j = i + _NSLOT - 1
            if j < nfull:
                if j >= _NSLOT:
                    wait_out(j % _NSLOT)
                issue_gather(j, j % _NSLOT)
            wait_gather(r)
            issue_out(i, r)
        for i in range(max(0, nfull - _NSLOT), nfull):
            wait_out(i % _NSLOT)

    return k(tbl_a, tbl_b, idx_a, idx_b)


def _sc_scatter_add(vals, idx, chunk, passes):
    _, ep, d = vals.shape
    es = ep // _NSUB
    nfull = es // _K
    sz = (chunk + 16) // _NSUB
    share = chunk // _NSUB
    zeros = jnp.zeros((sz, d), jnp.float32)
    nslot = 2 if passes > 1 else _NSLOT

    @functools.partial(
        pl.kernel,
        mesh=_mesh,
        out_type=jax.ShapeDtypeStruct((_B, passes * chunk, d), jnp.float32),
        scratch_types=[pltpu.VMEM((es,), jnp.int32),
                       pltpu.VMEM((nfull, _K), jnp.int32),
                       pltpu.VMEM((nslot, _K, d), jnp.float32),
                       pltpu.VMEM_SHARED((chunk + 16, d), jnp.float32),
                       pltpu.SemaphoreType.DMA((nslot,)),
                       pltpu.SemaphoreType.DMA((nslot,))],
        compiler_params=_sc_params,
    )
    def k(v_hbm, i_hbm, z_hbm, o_hbm, iva, lidx, vv, acc, sv, ss):
        c = lax.axis_index("c")
        s = lax.axis_index("s")
        base = s * es
        pltpu.sync_copy(i_hbm.at[pl.ds(base, es)], iva)

        def issue_load(i, r):
            pltpu.async_copy(v_hbm.at[c].at[pl.ds(base + i * _K, _K)],
                             vv.at[r], sv.at[r])

        def wait_load(r):
            pltpu.make_async_copy(v_hbm.at[c].at[pl.ds(0, _K)], vv.at[r],
                                  sv.at[r]).wait()

        def issue_scatter(i, r):
            pltpu.async_copy(vv.at[r], acc.at[lidx.at[i]], ss.at[r],
                             add=True)

        def wait_scatter(r):
            pltpu.make_async_copy(v_hbm.at[c].at[pl.ds(0, _K)], vv.at[r],
                                  ss.at[r]).wait()

        for p in range(passes):
            off0 = p * chunk
            pltpu.sync_copy(z_hbm, acc.at[pl.ds(s * sz, sz)])

            @pl.loop(0, nfull)
            def _(i):
                for j in range(_K // 16):
                    w = iva[pl.ds(i * _K + j * 16, 16)]
                    loc = w - off0
                    ok = (loc >= 0) & (loc < chunk)
                    lidx[i, pl.ds(j * 16, 16)] = jnp.where(ok, loc, chunk)

            plsc.subcore_barrier()
            for i in range(min(nslot - 1, nfull)):
                issue_load(i, i % nslot)
            for i in range(nfull):
                r = i % nslot
                j = i + nslot - 1
                if j < nfull:
                    if j >= nslot:
                        wait_scatter(j % nslot)
                    issue_load(j, j % nslot)
                wait_load(r)
                issue_scatter(i, r)
            for i in range(max(0, nfull - nslot), nfull):
                wait_scatter(i % nslot)

            plsc.subcore_barrier()
            pltpu.sync_copy(acc.at[pl.ds(s * share, share)],
                            o_hbm.at[c].at[pl.ds(off0 + s * share, share)])
            if p + 1 < passes:
                plsc.subcore_barrier()

    return k(vals, idx, zeros)


def _pad_rows(a, n):
    return jnp.concatenate(
        [a, jnp.zeros((n - a.shape[0],) + a.shape[1:], a.dtype)])


def _pad_idx(a, n, fill):
    return jnp.concatenate(
        [a.astype(jnp.int32), jnp.full((n - a.shape[0],), fill, jnp.int32)])


def _ceil_to(n, g):
    return -(-n // g) * g


def kernel(pndata, pn_coords, rn_coords, p2r_src, p2r_dst, p2r_edata,
           r2r_src, r2r_dst, r2r_edata, r2p_src, r2p_dst, r2p_edata, params):
    B, NP, DIN = pndata.shape
    NR = rn_coords.shape[0]
    E1, E2, E3 = p2r_src.shape[0], r2r_src.shape[0], r2p_src.shape[0]
    C = pn_coords.shape[1]
    E1p, E2p, E3p = (_ceil_to(e, _EPAD) for e in (E1, E2, E3))
    STEPS = 3

    p = params
    W1pn, b1pn, W2pn, b2pn = p['enc_pn']
    W1rn, b1rn, W2rn, b2rn = p['enc_rn']
    W1e1, b1e1, W2e1, b2e1 = p['enc_e']
    W1m1, b1m1, W2m1, b2m1 = p['enc_msg']
    W1n1, b1n1, W2n1, b2n1 = p['enc_node']
    W1e2, b1e2, W2e2, b2e2 = p['proc_e']
    W1m2, b1m2, W2m2, b2m2 = p['proc_msg']
    W1n2, b1n2, W2n2, b2n2 = p['proc_node']
    W1rd, b1rd, W2rd, b2rd = p['dec_rn']
    W1e3, b1e3, W2e3, b2e3 = p['dec_e']
    W1m3, b1m3, W2m3, b2m3 = p['dec_msg']
    W1n3, b1n3, W2n3, b2n3 = p['dec_node']
    W1do, b1do, W2do, b2do = p['dec_out']

    We1, Ws1, Wd1 = W1m1[:_D], W1m1[_D:2 * _D], W1m1[2 * _D:]
    We2, Ws2, Wd2 = W1m2[:_D], W1m2[_D:2 * _D], W1m2[2 * _D:]
    We3, Ws3, Wd3 = W1m3[:_D], W1m3[_D:2 * _D], W1m3[2 * _D:]
    Wa1, Wb1 = W1n1[:_D], W1n1[_D:]
    Wa2, Wb2 = W1n2[:_D], W1n2[_D:]
    Wa3, Wb3 = W1n3[:_D], W1n3[_D:]

    ed1 = _pad_rows(p2r_edata, E1p)
    ed2 = _pad_rows(r2r_edata, E2p)
    ed3 = _pad_rows(r2p_edata, E3p)
    src1 = _pad_idx(p2r_src, E1p, 0)
    dst1g = _pad_idx(p2r_dst, E1p, 0)
    dst1s = _pad_idx(p2r_dst, E1p, NR)
    src2 = _pad_idx(r2r_src, E2p, 0)
    dst2g = _pad_idx(r2r_dst, E2p, 0)
    dst2s = _pad_idx(r2r_dst, E2p, NR)
    src3 = _pad_idx(r2p_src, E3p, 0)
    dst3g = _pad_idx(r2p_dst, E3p, 0)
    dst3s = _pad_idx(r2p_dst, E3p, NP)

    CH_R = 5008
    CH_P = 20000

    pn, srcT1, dstT3 = _tc_mlp(
        [pn_coords, pndata.reshape(B * NP, DIN)], [W1pn[:C], W1pn[C:]],
        b1pn, W2pn, b2pn, out2=(Ws1, Wd3))
    rn0, dstT1 = _tc_mlp([rn_coords], [W1rn], b1rn, W2rn, b2rn, out2=(Wd1,))
    e1, eT1 = _tc_mlp([ed1], [W1e1], b1e1, W2e1, b2e1, out2=(We1,), block=2048)

    g_a, g_b = _sc_gather_pair(
        srcT1.reshape(B, NP, _D),
        jnp.broadcast_to(dstT1[None], (B, NR, _D)) + 0.0,
        src1, dst1g)
    e1n = _tc_mlp([eT1, g_a.reshape(B * E1p, _D), g_b.reshape(B * E1p, _D)],
                  [None, None, None], b1m1, W2m1, b2m1, res=e1, block=2048)
    agg = _sc_scatter_add(e1n.reshape(B, E1p, _D), dst1s, CH_R, 1)
    rn, srcTp, dstTp = _tc_mlp(
        [rn0, agg], [Wa1, Wb1], b1n1, W2n1, b2n1, res=rn0, out2=(Ws2, Wd2),
        nrows=B * NR)

    e2, eT2 = _tc_mlp([ed2], [W1e2], b1e2, W2e2, b2e2, out2=(We2,), block=2048)
    for t in range(STEPS):
        g_a, g_b = _sc_gather_pair(srcTp.reshape(B, NR, _D),
                                   dstTp.reshape(B, NR, _D), src2, dst2g)
        ga2 = g_a.reshape(B * E2p, _D)
        gb2 = g_b.reshape(B * E2p, _D)
        if t == 0:
            e2 = _tc_mlp([eT2, ga2, gb2], [None, None, None],
                         b1m2, W2m2, b2m2, res=e2, block=2048)
        else:
            e2 = _tc_mlp([e2, ga2, gb2], [We2, None, None],
                         b1m2, W2m2, b2m2, res=e2, block=2048)
        agg = _sc_scatter_add(e2.reshape(B, E2p, _D), dst2s, CH_R, 1)
        out2 = (Ws2, Wd2) if t + 1 < STEPS else ()
        r = _tc_mlp([rn, agg], [Wa2, Wb2], b1n2, W2n2, b2n2, res=rn,
                    out2=out2, nrows=B * NR)
        if out2:
            rn, srcTp, dstTp = r
        else:
            rn = r

    rnd, srcT3 = _tc_mlp([rn], [W1rd], b1rd, W2rd, b2rd, out2=(Ws3,))
    e3, eT3 = _tc_mlp([ed3], [W1e3], b1e3, W2e3, b2e3, out2=(We3,), block=2048)
    g_a, g_b = _sc_gather_pair(srcT3.reshape(B, NR, _D),
                               dstT3.reshape(B, NP, _D), src3, dst3g)
    e3n = _tc_mlp([eT3, g_a.reshape(B * E3p, _D), g_b.reshape(B * E3p, _D)],
                  [None, None, None], b1m3, W2m3, b2m3, res=e3, block=2048)
    aggp = _sc_scatter_add(e3n.reshape(B, E3p, _D), dst3s, CH_P, 3)
    pn1 = _tc_mlp([pn, aggp], [Wa3, Wb3], b1n3, W2n3, b2n3, res=pn,
                  nrows=B * NP)
    out = _tc_mlp([pn1], [W1do], b1do, W2do, b2do)
    return out.reshape(B, NP, -1)
